# baseline (speedup 1.0000x reference)
import jax
import jax.numpy as jnp
from jax import lax
from jax.experimental import pallas as pl
from jax.experimental.pallas import tpu as pltpu
from jax.experimental.pallas import tpu_sc as plsc

_NUM_CLASS = 80
_B = 16
_H = 128
_W = 128
_HW = _H * _W
_K = 100
_KPAD = 112
_NC = 2
_NS = 16
_L = 16
_CHUNK = 128
_NCHUNK = _HW // _CHUNK
_CONF_THRESH = 0.05
_BIG = 1 << 20


_CB = 80
_BB = 2


def _dense_body(hm_ref, conf_ref, cls_ref):
    c = pl.program_id(1)
    first = c == 0
    base = (c * _CB).astype(jnp.float32)
    neg_row = jnp.full((1, _W), -jnp.inf, jnp.float32)
    neg_col = jnp.full((_H, 1), -jnp.inf, jnp.float32)
    for bb in range(_BB):
        best = jnp.where(first, 0.0, conf_ref[bb])
        cls = jnp.where(first, 0.0, cls_ref[bb])
        for j in range(_CB):
            x = hm_ref[bb, j]
            up = jnp.concatenate([hm_ref[bb, j, pl.ds(1, _H - 1)], neg_row],
                                 axis=0)
            dn = jnp.concatenate([neg_row, hm_ref[bb, j, pl.ds(0, _H - 1)]],
                                 axis=0)
            m1 = jnp.maximum(jnp.maximum(up, dn), x)
            lf = jnp.concatenate([m1[:, 1:], neg_col], axis=1)
            rt = jnp.concatenate([neg_col, m1[:, :-1]], axis=1)
            pooled = jnp.maximum(jnp.maximum(lf, rt), m1)
            upd = (pooled == x) & (x > best)
            best = jnp.where(upd, x, best)
            cls = jnp.where(upd, base + j, cls)
        conf_ref[bb] = best
        cls_ref[bb] = cls


def _dense_pass(hm):
    return pl.pallas_call(
        _dense_body,
        grid=(_B // _BB, _NUM_CLASS // _CB),
        in_specs=[pl.BlockSpec((_BB, _CB, _H, _W), lambda b, c: (b, c, 0, 0))],
        out_specs=[
            pl.BlockSpec((_BB, _H, _W), lambda b, c: (b, 0, 0)),
            pl.BlockSpec((_BB, _H, _W), lambda b, c: (b, 0, 0)),
        ],
        out_shape=[
            jax.ShapeDtypeStruct((_B, _H, _W), jnp.float32),
            jax.ShapeDtypeStruct((_B, _H, _W), jnp.float32),
        ],
    )(hm)


def _sc_body(conf_hbm, cls_hbm, wh_hbm, reg_hbm, out_hbm,
             xv, gv, tvv, tiv, ig, gbuf, det, sem):
    wid = lax.axis_index("s") * _NC + lax.axis_index("c")

    @pl.when(wid < _B)
    def _():
        lane = lax.iota(jnp.int32, _L)
        lane0 = lane == 0

        pltpu.sync_copy(conf_hbm.at[wid], xv)

        zf = jnp.zeros((_L,), jnp.float32)
        zi = jnp.zeros((_L,), jnp.int32)
        for i in range(_KPAD // _L):
            tvv[pl.ds(i * _L, _L)] = zf
            tiv[pl.ds(i * _L, _L)] = zi

        def _chunk_max(base):
            v = xv[pl.ds(base, _L)]
            for j in range(1, _CHUNK // _L):
                v = jnp.maximum(v, xv[pl.ds(base + j * _L, _L)])
            return jnp.max(v)

        def _build(cc, carry):
            m = _chunk_max(cc * _CHUNK)
            plsc.store_scatter(gv, [jnp.full((_L,), cc, jnp.int32)],
                               jnp.full((_L,), m, jnp.float32), mask=lane0)
            return carry

        lax.fori_loop(0, _NCHUNK, _build, 0)

        def _extract(t, carry):
            gverts = [gv[pl.ds(k * _L, _L)] for k in range(_NCHUNK // _L)]
            vm = gverts[0]
            for k in range(1, len(gverts)):
                vm = jnp.maximum(vm, gverts[k])
            m = jnp.max(vm)
            mv = jnp.full((_L,), m, jnp.float32)

            cbest = jnp.full((_L,), _BIG, jnp.int32)
            for k in range(len(gverts)):
                f = plsc.all_reduce_ffs(gverts[k] == mv)
                cand = jnp.full((_L,), k * _L, jnp.int32) + f
                cbest = jnp.minimum(cbest, jnp.where(f < _L, cand, _BIG))
            cs = jnp.max(cbest)
            base = cs * _CHUNK

            pbest = jnp.full((_L,), _BIG, jnp.int32)
            for j in range(_CHUNK // _L):
                v = xv[pl.ds(base + j * _L, _L)]
                f = plsc.all_reduce_ffs(v == mv)
                cand = jnp.full((_L,), j * _L, jnp.int32) + f
                pbest = jnp.minimum(pbest, jnp.where(f < _L, cand, _BIG))
            pos = base + jnp.max(pbest)
            posv = jnp.full((_L,), pos, jnp.int32)

            tv = jnp.full((_L,), t, jnp.int32)
            plsc.store_scatter(tvv, [tv], mv, mask=lane0)
            plsc.store_scatter(tiv, [tv], posv, mask=lane0)

            plsc.store_scatter(xv, [posv],
                               jnp.full((_L,), -1.0, jnp.float32), mask=lane0)
            m2 = _chunk_max(base)
            plsc.store_scatter(gv, [jnp.full((_L,), cs, jnp.int32)],
                               jnp.full((_L,), m2, jnp.float32), mask=lane0)
            return carry

        lax.fori_loop(0, _K, _extract, 0)

        offs = [wid * _HW,
                wid * 2 * _HW, wid * 2 * _HW + _HW,
                wid * 2 * _HW, wid * 2 * _HW + _HW]
        for k in range(5):
            off = offs[k]
            for i in range(_KPAD // _L):
                ig[k, pl.ds(i * _L, _L)] = tiv[pl.ds(i * _L, _L)] + off
        tables = [cls_hbm, wh_hbm, wh_hbm, reg_hbm, reg_hbm]
        descs = [pltpu.async_copy(tables[k].at[ig.at[k]], gbuf.at[k], sem)
                 for k in range(5)]
        for d in descs:
            d.wait()

        inv = jnp.float32(1.0 / _W)
        for i in range(_KPAD // _L):
            ds_ = pl.ds(i * _L, _L)
            idx = tiv[ds_]
            conf = tvv[ds_]
            clsv = gbuf[0, ds_]
            w0 = gbuf[1, ds_]
            w1 = gbuf[2, ds_]
            r0 = gbuf[3, ds_]
            r1 = gbuf[4, ds_]
            xf = (idx & (_W - 1)).astype(jnp.float32)
            yf = (idx >> 7).astype(jnp.float32)
            xs = xf + r0
            ys = yf + r1
            hw = w0 * 0.5
            hh = w1 * 0.5
            x1 = (xs - hw) * inv
            y1 = (ys - hh) * inv
            x2 = (xs + hw) * inv
            y2 = (ys + hh) * inv
            sc = jnp.where(conf > _CONF_THRESH, conf, 0.0)
            slot = jnp.full((_L,), i * _L, jnp.int32) + lane
            bi = slot * 6
            mk = slot < _K
            plsc.store_scatter(det, [bi], x1, mask=mk)
            plsc.store_scatter(det, [bi + 1], y1, mask=mk)
            plsc.store_scatter(det, [bi + 2], x2, mask=mk)
            plsc.store_scatter(det, [bi + 3], y2, mask=mk)
            plsc.store_scatter(det, [bi + 4], sc, mask=mk)
            plsc.store_scatter(det, [bi + 5], clsv, mask=mk)

        pltpu.sync_copy(det, out_hbm.at[wid])


def _sc_pass(conf, cls_flat, wh_flat, reg_flat):
    mesh = plsc.VectorSubcoreMesh(core_axis_name="c", subcore_axis_name="s",
                                  num_cores=_NC, num_subcores=_NS)
    f = pl.kernel(
        _sc_body,
        out_type=jax.ShapeDtypeStruct((_B, _K * 6), jnp.float32),
        mesh=mesh,
        compiler_params=pltpu.CompilerParams(needs_layout_passes=False),
        scratch_types=[
            pltpu.VMEM((_HW,), jnp.float32),
            pltpu.VMEM((_NCHUNK,), jnp.float32),
            pltpu.VMEM((_KPAD,), jnp.float32),
            pltpu.VMEM((_KPAD,), jnp.int32),
            pltpu.VMEM((5, _KPAD), jnp.int32),
            pltpu.VMEM((5, _KPAD), jnp.float32),
            pltpu.VMEM((_K * 6,), jnp.float32),
            pltpu.SemaphoreType.DMA,
        ],
    )
    return f(conf, cls_flat, wh_flat, reg_flat)


def kernel(hm, wh, reg):
    conf, cls = _dense_pass(hm)
    det = _sc_pass(conf.reshape(_B, _HW),
                   cls.reshape(_B * _HW),
                   wh.reshape(_B * 2 * _HW),
                   reg.reshape(_B * 2 * _HW))
    return det.reshape(_B, _K, 6)


# speedup vs baseline: 1.0047x; 1.0047x over previous
import jax
import jax.numpy as jnp
from jax import lax
from jax.experimental import pallas as pl
from jax.experimental.pallas import tpu as pltpu
from jax.experimental.pallas import tpu_sc as plsc

_NUM_CLASS = 80
_B = 16
_H = 128
_W = 128
_HW = _H * _W
_K = 100
_KPAD = 112
_NC = 2
_NS = 16
_L = 16
_CHUNK = 128
_NCHUNK = _HW // _CHUNK
_CONF_THRESH = 0.05
_BIG = 1 << 20


_CB = 80
_BB = 2


def _dense_body(hm_ref, conf_ref, cls_ref):
    c = pl.program_id(1)
    first = c == 0
    base = (c * _CB).astype(jnp.float32)
    neg_row = jnp.full((1, _W), -jnp.inf, jnp.float32)
    neg_col = jnp.full((_H, 1), -jnp.inf, jnp.float32)
    for bb in range(_BB):
        best = jnp.where(first, 0.0, conf_ref[bb])
        cls = jnp.where(first, 0.0, cls_ref[bb])
        for j in range(_CB):
            x = hm_ref[bb, j]
            up = jnp.concatenate([x[1:], neg_row], axis=0)
            dn = jnp.concatenate([neg_row, x[:-1]], axis=0)
            m1 = jnp.maximum(jnp.maximum(up, dn), x)
            lf = jnp.concatenate([m1[:, 1:], neg_col], axis=1)
            rt = jnp.concatenate([neg_col, m1[:, :-1]], axis=1)
            pooled = jnp.maximum(jnp.maximum(lf, rt), m1)
            upd = (pooled == x) & (x > best)
            best = jnp.where(upd, x, best)
            cls = jnp.where(upd, base + j, cls)
        conf_ref[bb] = best
        cls_ref[bb] = cls


def _dense_pass(hm):
    return pl.pallas_call(
        _dense_body,
        grid=(_B // _BB, _NUM_CLASS // _CB),
        in_specs=[pl.BlockSpec((_BB, _CB, _H, _W), lambda b, c: (b, c, 0, 0))],
        out_specs=[
            pl.BlockSpec((_BB, _H, _W), lambda b, c: (b, 0, 0)),
            pl.BlockSpec((_BB, _H, _W), lambda b, c: (b, 0, 0)),
        ],
        out_shape=[
            jax.ShapeDtypeStruct((_B, _H, _W), jnp.float32),
            jax.ShapeDtypeStruct((_B, _H, _W), jnp.float32),
        ],
    )(hm)


def _sc_body(conf_hbm, cls_hbm, wh_hbm, reg_hbm, out_hbm,
             xv, gv, tvv, tiv, ig, gbuf, det, sem):
    wid = lax.axis_index("s") * _NC + lax.axis_index("c")

    @pl.when(wid < _B)
    def _():
        lane = lax.iota(jnp.int32, _L)
        lane0 = lane == 0

        pltpu.sync_copy(conf_hbm.at[wid], xv)

        zf = jnp.zeros((_L,), jnp.float32)
        zi = jnp.zeros((_L,), jnp.int32)
        for i in range(_KPAD // _L):
            tvv[pl.ds(i * _L, _L)] = zf
            tiv[pl.ds(i * _L, _L)] = zi

        def _chunk_max(base):
            v = xv[pl.ds(base, _L)]
            for j in range(1, _CHUNK // _L):
                v = jnp.maximum(v, xv[pl.ds(base + j * _L, _L)])
            return jnp.max(v)

        def _build(cc, carry):
            m = _chunk_max(cc * _CHUNK)
            plsc.store_scatter(gv, [jnp.full((_L,), cc, jnp.int32)],
                               jnp.full((_L,), m, jnp.float32), mask=lane0)
            return carry

        lax.fori_loop(0, _NCHUNK, _build, 0)

        def _extract(t, carry):
            gverts = [gv[pl.ds(k * _L, _L)] for k in range(_NCHUNK // _L)]
            vm = gverts[0]
            for k in range(1, len(gverts)):
                vm = jnp.maximum(vm, gverts[k])
            m = jnp.max(vm)
            mv = jnp.full((_L,), m, jnp.float32)

            cbest = jnp.full((_L,), _BIG, jnp.int32)
            for k in range(len(gverts)):
                f = plsc.all_reduce_ffs(gverts[k] == mv)
                cand = jnp.full((_L,), k * _L, jnp.int32) + f
                cbest = jnp.minimum(cbest, jnp.where(f < _L, cand, _BIG))
            basev = cbest * _CHUNK

            pbest = jnp.full((_L,), _BIG, jnp.int32)
            for j in range(_CHUNK // _L):
                v = plsc.load_gather(xv, [basev + (j * _L) + lane])
                f = plsc.all_reduce_ffs(v == mv)
                cand = jnp.full((_L,), j * _L, jnp.int32) + f
                pbest = jnp.minimum(pbest, jnp.where(f < _L, cand, _BIG))
            posv = basev + pbest

            tv = jnp.full((_L,), t, jnp.int32)
            plsc.store_scatter(tvv, [tv], mv, mask=lane0)
            plsc.store_scatter(tiv, [tv], posv, mask=lane0)

            plsc.store_scatter(xv, [posv],
                               jnp.full((_L,), -1.0, jnp.float32), mask=lane0)
            v = plsc.load_gather(xv, [basev + lane])
            for j in range(1, _CHUNK // _L):
                v = jnp.maximum(v, plsc.load_gather(xv, [basev + (j * _L) + lane]))
            m2 = jnp.max(v)
            plsc.store_scatter(gv, [cbest],
                               jnp.full((_L,), m2, jnp.float32), mask=lane0)
            return carry

        lax.fori_loop(0, _K, _extract, 0)

        offs = [wid * _HW,
                wid * 2 * _HW, wid * 2 * _HW + _HW,
                wid * 2 * _HW, wid * 2 * _HW + _HW]
        for k in range(5):
            off = offs[k]
            for i in range(_KPAD // _L):
                ig[k, pl.ds(i * _L, _L)] = tiv[pl.ds(i * _L, _L)] + off
        tables = [cls_hbm, wh_hbm, wh_hbm, reg_hbm, reg_hbm]
        descs = [pltpu.async_copy(tables[k].at[ig.at[k]], gbuf.at[k], sem)
                 for k in range(5)]
        for d in descs:
            d.wait()

        inv = jnp.float32(1.0 / _W)
        for i in range(_KPAD // _L):
            ds_ = pl.ds(i * _L, _L)
            idx = tiv[ds_]
            conf = tvv[ds_]
            clsv = gbuf[0, ds_]
            w0 = gbuf[1, ds_]
            w1 = gbuf[2, ds_]
            r0 = gbuf[3, ds_]
            r1 = gbuf[4, ds_]
            xf = (idx & (_W - 1)).astype(jnp.float32)
            yf = (idx >> 7).astype(jnp.float32)
            xs = xf + r0
            ys = yf + r1
            hw = w0 * 0.5
            hh = w1 * 0.5
            x1 = (xs - hw) * inv
            y1 = (ys - hh) * inv
            x2 = (xs + hw) * inv
            y2 = (ys + hh) * inv
            sc = jnp.where(conf > _CONF_THRESH, conf, 0.0)
            slot = jnp.full((_L,), i * _L, jnp.int32) + lane
            bi = slot * 6
            mk = slot < _K
            plsc.store_scatter(det, [bi], x1, mask=mk)
            plsc.store_scatter(det, [bi + 1], y1, mask=mk)
            plsc.store_scatter(det, [bi + 2], x2, mask=mk)
            plsc.store_scatter(det, [bi + 3], y2, mask=mk)
            plsc.store_scatter(det, [bi + 4], sc, mask=mk)
            plsc.store_scatter(det, [bi + 5], clsv, mask=mk)

        pltpu.sync_copy(det, out_hbm.at[wid])


def _sc_pass(conf, cls_flat, wh_flat, reg_flat):
    mesh = plsc.VectorSubcoreMesh(core_axis_name="c", subcore_axis_name="s",
                                  num_cores=_NC, num_subcores=_NS)
    f = pl.kernel(
        _sc_body,
        out_type=jax.ShapeDtypeStruct((_B, _K * 6), jnp.float32),
        mesh=mesh,
        compiler_params=pltpu.CompilerParams(needs_layout_passes=False),
        scratch_types=[
            pltpu.VMEM((_HW,), jnp.float32),
            pltpu.VMEM((_NCHUNK,), jnp.float32),
            pltpu.VMEM((_KPAD,), jnp.float32),
            pltpu.VMEM((_KPAD,), jnp.int32),
            pltpu.VMEM((5, _KPAD), jnp.int32),
            pltpu.VMEM((5, _KPAD), jnp.float32),
            pltpu.VMEM((_K * 6,), jnp.float32),
            pltpu.SemaphoreType.DMA,
        ],
    )
    return f(conf, cls_flat, wh_flat, reg_flat)


def kernel(hm, wh, reg):
    conf, cls = _dense_pass(hm)
    det = _sc_pass(conf.reshape(_B, _HW),
                   cls.reshape(_B * _HW),
                   wh.reshape(_B * 2 * _HW),
                   reg.reshape(_B * 2 * _HW))
    return det.reshape(_B, _K, 6)
